# baseline (speedup 1.0000x reference)
import functools

import jax
import jax.numpy as jnp
from jax import lax
from jax.experimental import pallas as pl
from jax.experimental.pallas import tpu as pltpu
from jax.experimental.pallas import tpu_sc as plsc

N = 10000
NPAD = 10240
D = 128
HW = 64
AW = 80
E = 320000
ALPHA = 0.2
C = 128
NCH_ALL = E // C
NCH_BASE = NCH_ALL // 16
NCH_REM = NCH_ALL % 16
RPT = NPAD // 16
ZR = 128


def _prep_body(h_ref, w_ref, a_ref, wh_ref, s1_ref, s2_ref):
    wh = jnp.dot(h_ref[...], w_ref[...], preferred_element_type=jnp.float32)
    wh_ref[0] = wh[:, :HW]
    wh_ref[1] = wh[:, HW:]
    s1_ref[...] = jnp.dot(wh, a_ref[:D, :], preferred_element_type=jnp.float32)
    s2_ref[...] = jnp.dot(wh, a_ref[D:, :], preferred_element_type=jnp.float32)


def _prep(h_pad, W, a):
    blk = 1280
    return pl.pallas_call(
        _prep_body,
        grid=(NPAD // blk,),
        in_specs=[
            pl.BlockSpec((blk, D), lambda i: (i, 0)),
            pl.BlockSpec((D, D), lambda i: (0, 0)),
            pl.BlockSpec((2 * D, 1), lambda i: (0, 0)),
        ],
        out_specs=[
            pl.BlockSpec((2, blk, HW), lambda i: (0, i, 0)),
            pl.BlockSpec((blk, 1), lambda i: (i, 0)),
            pl.BlockSpec((blk, 1), lambda i: (i, 0)),
        ],
        out_shape=[
            jax.ShapeDtypeStruct((2, NPAD, HW), jnp.float32),
            jax.ShapeDtypeStruct((NPAD, 1), jnp.float32),
            jax.ShapeDtypeStruct((NPAD, 1), jnp.float32),
        ],
    )(h_pad, W, a)


@functools.partial(
    pl.kernel,
    out_type=jax.ShapeDtypeStruct((2, NPAD, AW), jnp.float32),
    mesh=plsc.VectorSubcoreMesh(core_axis_name="c", subcore_axis_name="s"),
    compiler_params=pltpu.CompilerParams(
        needs_layout_passes=False, use_tc_tiling_on_sc=False),
    scratch_types=[
        pltpu.VMEM((NPAD,), jnp.float32),
        pltpu.VMEM((NPAD,), jnp.float32),
        [pltpu.VMEM((2, C), jnp.int32) for _ in range(2)],
        [pltpu.VMEM((C,), jnp.int32) for _ in range(2)],
        [pltpu.VMEM((C,), jnp.int32) for _ in range(2)],
        [pltpu.VMEM((C, HW), jnp.float32) for _ in range(2)],
        [pltpu.VMEM((C, AW), jnp.float32) for _ in range(2)],
        pltpu.VMEM((16,), jnp.float32),
        pltpu.VMEM((ZR, AW), jnp.float32),
        pltpu.VMEM_SHARED((NPAD, AW), jnp.float32),
        [pltpu.SemaphoreType.DMA for _ in range(2)],
        [pltpu.SemaphoreType.DMA for _ in range(2)],
        [pltpu.SemaphoreType.DMA for _ in range(2)],
    ],
)
def _sc_kernel(wh_hbm, s1_hbm, s2_hbm, ei_hbm, part_hbm,
               s1_v, s2_v, idx_v, col2_v, rowc_v, rows_v,
               scaled_v, wbuf, zbuf, acc, gsem, isem, ssem):
    cid = lax.axis_index("c")
    sid = lax.axis_index("s")
    nct = NCH_BASE + jnp.where(sid < NCH_REM, 1, 0)
    half_base = cid * NPAD

    pltpu.sync_copy(s1_hbm, s1_v)
    pltpu.sync_copy(s2_hbm, s2_v)

    def _mx(i, m):
        return jnp.maximum(m, s2_v[pl.ds(i * 16, 16)])
    gmax = lax.fori_loop(
        0, NPAD // 16, _mx, jnp.full((16,), -jnp.inf, jnp.float32))
    ii16 = lax.iota(jnp.int32, 16)
    for sh in (8, 4, 2, 1):
        wbuf[...] = gmax
        gmax = jnp.maximum(
            gmax, plsc.load_gather(wbuf, [jnp.bitwise_xor(ii16, sh)]))

    def _zrow(i, carry):
        for cb in range(AW // 16):
            zbuf[i, pl.ds(cb * 16, 16)] = jnp.zeros((16,), jnp.float32)
        return carry
    lax.fori_loop(0, ZR, _zrow, 0)
    for b in range(RPT // ZR):
        pltpu.sync_copy(zbuf, acc.at[pl.ds(sid * RPT + b * ZR, ZR)])
    def _zpad(r, carry):
        for b in range(2):
            scaled_v[b][r, pl.ds(HW, 16)] = jnp.zeros((16,), jnp.float32)
        return carry
    lax.fori_loop(0, C, _zpad, 0)
    plsc.subcore_barrier()

    def _eoff(t):
        return (sid + 16 * t) * C

    def _start_idx(t, b):
        pltpu.async_copy(
            ei_hbm.at[:, pl.ds(_eoff(t), C)], idx_v[b], isem[b])

    def _start_gather(b):
        for k in range(C // 16):
            col2_v[b][pl.ds(k * 16, 16)] = (
                idx_v[b][1, pl.ds(k * 16, 16)] + half_base)
        pltpu.async_copy(wh_hbm.at[col2_v[b]], rows_v[b], gsem[b])

    pltpu.sync_copy(ei_hbm.at[:, pl.ds(sid * C, C)], idx_v[0])
    _start_gather(0)
    _start_idx(1, 1)

    dnums = lax.GatherDimensionNumbers(
        offset_dims=(), collapsed_slice_dims=(0,), start_index_map=(0,))

    def _do_chunk(t, b):
        @pl.when(t + 1 < nct)
        def _():
            pltpu.make_async_copy(
                ei_hbm.at[:, pl.ds(_eoff(t + 1), C)],
                idx_v[1 - b], isem[1 - b]).wait()
            _start_gather(1 - b)
        pltpu.make_async_copy(wh_hbm.at[col2_v[b]], rows_v[b], gsem[b]).wait()

        @pl.when(t >= 2)
        def _():
            pltpu.make_async_copy(
                scaled_v[b], acc.at[rowc_v[b]], ssem[b]).wait()

        def _grp(k):
            r16 = idx_v[b][0, pl.ds(k * 16, 16)]
            rowc_v[b][pl.ds(k * 16, 16)] = r16
            c16 = idx_v[b][1, pl.ds(k * 16, 16)]
            s1g = plsc.load_gather(s1_v, [r16])
            s2g = plsc.load_gather(s2_v, [c16])
            e = s1g + s2g
            e = jnp.where(e > 0, e, ALPHA * e)
            m = s1g + gmax
            m = jnp.where(m > 0, m, ALPHA * m)
            w = jnp.exp(e - m)
            plsc.store_scatter(
                scaled_v[b], [k * 16 + ii16, jnp.full((16,), HW, jnp.int32)],
                w)
            for j in range(16):
                wj = lax.gather(
                    w, jnp.full((16, 1), j, jnp.int32), dnums, (1,),
                    mode=lax.GatherScatterMode.PROMISE_IN_BOUNDS)
                r = k * 16 + j
                for cb in range(HW // 16):
                    scaled_v[b][r, pl.ds(cb * 16, 16)] = (
                        rows_v[b][r, pl.ds(cb * 16, 16)] * wj)
        for k in range(C // 16):
            _grp(k)
        pltpu.async_copy(scaled_v[b], acc.at[rowc_v[b]], ssem[b], add=True)

        @pl.when(t + 2 < nct)
        def _():
            _start_idx(t + 2, b)

    def _pair(i, carry):
        _do_chunk(2 * i, 0)
        _do_chunk(2 * i + 1, 1)
        return carry
    lax.fori_loop(0, NCH_BASE // 2, _pair, 0)

    @pl.when(sid < NCH_REM)
    def _():
        _do_chunk(NCH_BASE, 0)
    for b in range(2):
        pltpu.make_async_copy(scaled_v[b], acc.at[rowc_v[b]], ssem[b]).wait()
    plsc.subcore_barrier()

    for b in range(RPT // ZR):
        r0 = sid * RPT + b * ZR
        pltpu.sync_copy(acc.at[pl.ds(r0, ZR)],
                        part_hbm.at[cid, pl.ds(r0, ZR)])


def _fin_body(p_ref, o_ref):
    halves = []
    for c in range(2):
        num = p_ref[c, :, :HW]
        den = p_ref[c, :, HW:HW + 1]
        hp = num / den
        act = jnp.where(hp > 0, hp, jnp.exp(hp) - 1.0)
        halves.append(jnp.where(den > 0, act, 0.0))
    o_ref[...] = jnp.concatenate(halves, axis=1)


def _fin(part):
    blk = 1000
    return pl.pallas_call(
        _fin_body,
        grid=(N // blk,),
        in_specs=[pl.BlockSpec((2, blk, AW), lambda i: (0, i, 0))],
        out_specs=pl.BlockSpec((blk, D), lambda i: (i, 0)),
        out_shape=jax.ShapeDtypeStruct((N, D), jnp.float32),
    )(part)


def kernel(h, edge_index, W, a):
    h32 = h.astype(jnp.float32)
    h_pad = jnp.pad(h32, ((0, NPAD - N), (0, 0)))
    ei32 = edge_index.astype(jnp.int32)
    wh2, s1, s2 = _prep(h_pad, W.astype(jnp.float32), a.astype(jnp.float32))
    wh_stack = wh2.reshape(2 * NPAD, HW)
    part = _sc_kernel(wh_stack, s1.reshape(NPAD), s2.reshape(NPAD), ei32)
    return _fin(part)


# speedup vs baseline: 1.1950x; 1.1950x over previous
import functools

import jax
import jax.numpy as jnp
from jax import lax
from jax.experimental import pallas as pl
from jax.experimental.pallas import tpu as pltpu
from jax.experimental.pallas import tpu_sc as plsc

N = 10000
NPAD = 10240
D = 128
HW = 64
AW = 80
E = 320000
ALPHA = 0.2
EPS = E // 16
C = 80
NCHUNK = EPS // C
RPT = NPAD // 16
ZR = 128


def _prep_body(h_ref, w_ref, a_ref, wh_ref, s1_ref, s2_ref):
    wh = jnp.dot(h_ref[...], w_ref[...], preferred_element_type=jnp.float32)
    wh_ref[0] = wh[:, :HW]
    wh_ref[1] = wh[:, HW:]
    s1_ref[...] = jnp.dot(wh, a_ref[:D, :], preferred_element_type=jnp.float32)
    s2_ref[...] = jnp.dot(wh, a_ref[D:, :], preferred_element_type=jnp.float32)


def _prep(h_pad, W, a):
    blk = 1280
    return pl.pallas_call(
        _prep_body,
        grid=(NPAD // blk,),
        in_specs=[
            pl.BlockSpec((blk, D), lambda i: (i, 0)),
            pl.BlockSpec((D, D), lambda i: (0, 0)),
            pl.BlockSpec((2 * D, 1), lambda i: (0, 0)),
        ],
        out_specs=[
            pl.BlockSpec((2, blk, HW), lambda i: (0, i, 0)),
            pl.BlockSpec((blk, 1), lambda i: (i, 0)),
            pl.BlockSpec((blk, 1), lambda i: (i, 0)),
        ],
        out_shape=[
            jax.ShapeDtypeStruct((2, NPAD, HW), jnp.float32),
            jax.ShapeDtypeStruct((NPAD, 1), jnp.float32),
            jax.ShapeDtypeStruct((NPAD, 1), jnp.float32),
        ],
    )(h_pad, W, a)


@functools.partial(
    pl.kernel,
    out_type=jax.ShapeDtypeStruct((2, NPAD, AW), jnp.float32),
    mesh=plsc.VectorSubcoreMesh(core_axis_name="c", subcore_axis_name="s"),
    compiler_params=pltpu.CompilerParams(
        needs_layout_passes=False, use_tc_tiling_on_sc=False),
    scratch_types=[
        pltpu.VMEM((NPAD,), jnp.float32),
        pltpu.VMEM((NPAD,), jnp.float32),
        [pltpu.VMEM((2, C), jnp.int32) for _ in range(4)],
        [pltpu.VMEM((C,), jnp.int32) for _ in range(4)],
        [pltpu.VMEM((C,), jnp.int32) for _ in range(2)],
        [pltpu.VMEM((C, HW), jnp.float32) for _ in range(4)],
        [pltpu.VMEM((C, AW), jnp.float32) for _ in range(2)],
        pltpu.VMEM((16,), jnp.float32),
        pltpu.VMEM((ZR, AW), jnp.float32),
        pltpu.VMEM_SHARED((NPAD, AW), jnp.float32),
        [pltpu.SemaphoreType.DMA for _ in range(4)],
        [pltpu.SemaphoreType.DMA for _ in range(4)],
        [pltpu.SemaphoreType.DMA for _ in range(2)],
    ],
)
def _sc_kernel(wh_hbm, s1_hbm, s2_hbm, ei_hbm, part_hbm,
               s1_v, s2_v, idx_v, col2_v, rowc_v, rows_v,
               scaled_v, wbuf, zbuf, acc, gsem, isem, ssem):
    cid = lax.axis_index("c")
    sid = lax.axis_index("s")
    eoff = sid * EPS
    half_base = cid * NPAD

    pltpu.sync_copy(s1_hbm, s1_v)
    pltpu.sync_copy(s2_hbm, s2_v)

    def _mx(i, m):
        return jnp.maximum(m, s2_v[pl.ds(i * 16, 16)])
    gmax = lax.fori_loop(
        0, NPAD // 16, _mx, jnp.full((16,), -jnp.inf, jnp.float32))
    ii16 = lax.iota(jnp.int32, 16)
    for sh in (8, 4, 2, 1):
        wbuf[...] = gmax
        gmax = jnp.maximum(
            gmax, plsc.load_gather(wbuf, [jnp.bitwise_xor(ii16, sh)]))

    def _zrow(i, carry):
        for cb in range(AW // 16):
            zbuf[i, pl.ds(cb * 16, 16)] = jnp.zeros((16,), jnp.float32)
        return carry
    lax.fori_loop(0, ZR, _zrow, 0)
    for b in range(RPT // ZR):
        pltpu.sync_copy(zbuf, acc.at[pl.ds(sid * RPT + b * ZR, ZR)])
    def _zpad(r, carry):
        for b in range(2):
            scaled_v[b][r, pl.ds(HW, 16)] = jnp.zeros((16,), jnp.float32)
        return carry
    lax.fori_loop(0, C, _zpad, 0)
    plsc.subcore_barrier()

    def _start_idx(t, b):
        pltpu.async_copy(
            ei_hbm.at[:, pl.ds(eoff + t * C, C)], idx_v[b], isem[b])

    def _start_gather(b):
        for k in range(C // 16):
            col2_v[b][pl.ds(k * 16, 16)] = (
                idx_v[b][1, pl.ds(k * 16, 16)] + half_base)
        pltpu.async_copy(wh_hbm.at[col2_v[b]], rows_v[b], gsem[b])

    pltpu.sync_copy(ei_hbm.at[:, pl.ds(eoff, C)], idx_v[0])
    for u in range(1, 4):
        _start_idx(u, u)
    _start_gather(0)
    pltpu.make_async_copy(
        ei_hbm.at[:, pl.ds(eoff + C, C)], idx_v[1], isem[1]).wait()
    _start_gather(1)

    dnums = lax.GatherDimensionNumbers(
        offset_dims=(), collapsed_slice_dims=(0,), start_index_map=(0,))

    def _do_chunk(t, b):
        b2 = (b + 2) % 4
        @pl.when(t + 2 < NCHUNK)
        def _():
            pltpu.make_async_copy(
                ei_hbm.at[:, pl.ds(eoff + (t + 2) * C, C)],
                idx_v[b2], isem[b2]).wait()
            _start_gather(b2)
        pltpu.make_async_copy(wh_hbm.at[col2_v[b]], rows_v[b], gsem[b]).wait()

        sb = b % 2
        @pl.when(t >= 2)
        def _():
            pltpu.make_async_copy(
                scaled_v[sb], acc.at[rowc_v[sb]], ssem[sb]).wait()

        def _grp(k):
            r16 = idx_v[b][0, pl.ds(k * 16, 16)]
            rowc_v[sb][pl.ds(k * 16, 16)] = r16
            c16 = idx_v[b][1, pl.ds(k * 16, 16)]
            s1g = plsc.load_gather(s1_v, [r16])
            s2g = plsc.load_gather(s2_v, [c16])
            e = s1g + s2g
            e = jnp.where(e > 0, e, ALPHA * e)
            m = s1g + gmax
            m = jnp.where(m > 0, m, ALPHA * m)
            w = jnp.exp(e - m)
            plsc.store_scatter(
                scaled_v[sb], [k * 16 + ii16, jnp.full((16,), HW, jnp.int32)],
                w)
            for j in range(16):
                wj = lax.gather(
                    w, jnp.full((16, 1), j, jnp.int32), dnums, (1,),
                    mode=lax.GatherScatterMode.PROMISE_IN_BOUNDS)
                r = k * 16 + j
                for cb in range(HW // 16):
                    scaled_v[sb][r, pl.ds(cb * 16, 16)] = (
                        rows_v[b][r, pl.ds(cb * 16, 16)] * wj)
        for k in range(C // 16):
            _grp(k)
        pltpu.async_copy(scaled_v[sb], acc.at[rowc_v[sb]], ssem[sb], add=True)

        @pl.when(t + 4 < NCHUNK)
        def _():
            _start_idx(t + 4, b)

    def _quad(i, carry):
        for u in range(4):
            _do_chunk(4 * i + u, u)
        return carry
    lax.fori_loop(0, NCHUNK // 4, _quad, 0)
    for u in range(NCHUNK % 4):
        _do_chunk(NCHUNK - (NCHUNK % 4) + u, u)
    for sb in range(2):
        pltpu.make_async_copy(scaled_v[sb], acc.at[rowc_v[sb]], ssem[sb]).wait()
    plsc.subcore_barrier()

    for b in range(RPT // ZR):
        r0 = sid * RPT + b * ZR
        pltpu.sync_copy(acc.at[pl.ds(r0, ZR)],
                        part_hbm.at[cid, pl.ds(r0, ZR)])


def _fin_body(p_ref, o_ref):
    halves = []
    for c in range(2):
        num = p_ref[c, :, :HW]
        den = p_ref[c, :, HW:HW + 1]
        hp = num / den
        act = jnp.where(hp > 0, hp, jnp.exp(hp) - 1.0)
        halves.append(jnp.where(den > 0, act, 0.0))
    o_ref[...] = jnp.concatenate(halves, axis=1)


def _fin(part):
    blk = 1000
    return pl.pallas_call(
        _fin_body,
        grid=(N // blk,),
        in_specs=[pl.BlockSpec((2, blk, AW), lambda i: (0, i, 0))],
        out_specs=pl.BlockSpec((blk, D), lambda i: (i, 0)),
        out_shape=jax.ShapeDtypeStruct((N, D), jnp.float32),
    )(part)


def kernel(h, edge_index, W, a):
    h32 = h.astype(jnp.float32)
    h_pad = jnp.pad(h32, ((0, NPAD - N), (0, 0)))
    ei32 = edge_index.astype(jnp.int32)
    wh2, s1, s2 = _prep(h_pad, W.astype(jnp.float32), a.astype(jnp.float32))
    wh_stack = wh2.reshape(2 * NPAD, HW)
    part = _sc_kernel(wh_stack, s1.reshape(NPAD), s2.reshape(NPAD), ei32)
    return _fin(part)
